# Initial kernel scaffold; baseline (speedup 1.0000x reference)
import jax
import jax.numpy as jnp
from jax.experimental import pallas as pl


def kernel(rpn_cls_logits, rpn_bbox_pred, gt_boxes, gt_labels, feat_map_shape):
    raise NotImplementedError("write your pallas kernel here")


# speedup vs baseline: 2.6965x; 2.6965x over previous
import jax
import jax.numpy as jnp
import numpy as np
from jax.experimental import pallas as pl

NA = 9
FH = 64
FW = 64
B = 4
NG = 50
A = FH * FW * NA
AR = A // 128
GP = 64
R1 = 16
NJ = AR // R1

_INTERPRET = False


def _anchors_np():
    base_size = 16
    anchors = []
    cx = base_size / 2.0
    cy = base_size / 2.0
    for r in (0.5, 1.0, 2.0):
        for s in (8, 16, 32):
            area = float(base_size * s) ** 2
            w = np.sqrt(area / r)
            h = w * r
            anchors.append([cx - 0.5 * w, cy - 0.5 * h, cx + 0.5 * w, cy + 0.5 * h])
    base = np.array(anchors, dtype=np.float32)
    shift_x = np.arange(FW, dtype=np.float32) * 16.0
    shift_y = np.arange(FH, dtype=np.float32) * 16.0
    sx, sy = np.meshgrid(shift_x, shift_y, indexing="ij")
    shifts = np.stack([sx, sy, sx, sy], axis=-1).reshape(-1, 4).astype(np.float32)
    return (base[None, :, :] + shifts[:, None, :]).reshape(-1, 4).astype(np.float32)


_ANCH = _anchors_np()
_AX = [jnp.asarray(_ANCH[:, c].reshape(AR, 128)) for c in range(4)]


def _s1_body(ax1r, ay1r, ax2r, ay2r, g0r, g1r, g2r, g3r, vmr,
             mir, m0r, m1r, m2r, m3r):
    ax1 = ax1r[...][None]
    ay1 = ay1r[...][None]
    ax2 = ax2r[...][None]
    ay2 = ay2r[...][None]
    gx1 = g0r[0]
    gy1 = g1r[0]
    gx2 = g2r[0]
    gy2 = g3r[0]
    vm = vmr[0]
    x1 = jnp.maximum(ax1, gx1)
    y1 = jnp.maximum(ay1, gy1)
    x2 = jnp.minimum(ax2, gx2)
    y2 = jnp.minimum(ay2, gy2)
    inter = jnp.maximum(0.0, x2 - x1) * jnp.maximum(0.0, y2 - y1)
    a1 = (ax2 - ax1) * (ay2 - ay1)
    a2 = (gx2 - gx1) * (gy2 - gy1)
    iou = inter / (a1 + a2 - inter + 1e-8)
    iou = iou * vm + (vm - 1.0)
    mx = jnp.max(iou, axis=0)
    it = jax.lax.broadcasted_iota(jnp.int32, (GP, R1, 128), 0)
    am = jnp.min(jnp.where(iou >= mx[None], it, GP), axis=0)
    oh = (it == am[None]).astype(jnp.float32)
    mir[0] = mx
    m0r[0] = jnp.sum(oh * gx1, axis=0)
    m1r[0] = jnp.sum(oh * gy1, axis=0)
    m2r[0] = jnp.sum(oh * gx2, axis=0)
    m3r[0] = jnp.sum(oh * gy2, axis=0)


def _stage1(g, vm):
    anch_spec = pl.BlockSpec((R1, 128), lambda b, j: (j, 0))
    gt_spec = pl.BlockSpec((1, GP, 1, 1), lambda b, j: (b, 0, 0, 0))
    out_spec = pl.BlockSpec((1, R1, 128), lambda b, j: (b, j, 0))
    shp = jax.ShapeDtypeStruct((B, AR, 128), jnp.float32)
    return pl.pallas_call(
        _s1_body,
        grid=(B, NJ),
        in_specs=[anch_spec] * 4 + [gt_spec] * 5,
        out_specs=[out_spec] * 5,
        out_shape=[shp] * 5,
        interpret=_INTERPRET,
    )(*_AX, *g, vm)


def _smooth_l1(d):
    ad = jnp.abs(d)
    return jnp.where(ad < 1.0, 0.5 * d * d, ad - 0.5)


def _s2_body(mir, m0r, m1r, m2r, m3r, l0r, l1r, p0r, p1r, p2r, p3r,
             ax1r, ay1r, ax2r, ay2r, labr, tgtr, cer, slr, cntr):
    b = pl.program_id(0)
    mx = mir[0]
    lab = mx >= 0.7
    labf = lab.astype(jnp.float32)
    labr[0] = lab.astype(jnp.int32)
    ax1 = ax1r[...]
    ay1 = ay1r[...]
    ax2 = ax2r[...]
    ay2 = ay2r[...]
    bw = ax2 - ax1 + 1.0
    bh = ay2 - ay1 + 1.0
    bcx = ax1 + 0.5 * bw
    bcy = ay1 + 0.5 * bh
    m0 = m0r[0]
    m1 = m1r[0]
    m2 = m2r[0]
    m3 = m3r[0]
    gw = m2 - m0 + 1.0
    gh = m3 - m1 + 1.0
    gcx = m0 + 0.5 * gw
    gcy = m1 + 0.5 * gh
    t0 = (gcx - bcx) / bw
    t1 = (gcy - bcy) / bh
    t2 = jnp.log(gw / bw)
    t3 = jnp.log(gh / bh)
    tgtr[0, 0] = t0
    tgtr[0, 1] = t1
    tgtr[0, 2] = t2
    tgtr[0, 3] = t3
    l0 = l0r[0]
    l1 = l1r[0]
    mm = jnp.maximum(l0, l1)
    lse = mm + jnp.log(jnp.exp(l0 - mm) + jnp.exp(l1 - mm))
    ce_b = jnp.sum(lse - jnp.where(lab, l1, l0), keepdims=True)
    sl = (_smooth_l1(p0r[0] - t0) + _smooth_l1(p1r[0] - t1)
          + _smooth_l1(p2r[0] - t2) + _smooth_l1(p3r[0] - t3))
    sl_b = jnp.sum(sl * labf, keepdims=True)
    cnt_b = jnp.sum(labf, keepdims=True)

    @pl.when(b == 0)
    def _():
        cer[...] = jnp.zeros((1, 1), jnp.float32)
        slr[...] = jnp.zeros((1, 1), jnp.float32)
        cntr[...] = jnp.zeros((1, 1), jnp.float32)

    cer[...] += ce_b
    slr[...] += sl_b
    cntr[...] += cnt_b


def _stage2(mi, m, l0, l1, p):
    big = pl.BlockSpec((1, AR, 128), lambda b: (b, 0, 0))
    anch_spec = pl.BlockSpec((AR, 128), lambda b: (0, 0))
    scal = pl.BlockSpec((1, 1), lambda b: (0, 0))
    return pl.pallas_call(
        _s2_body,
        grid=(B,),
        in_specs=[big] * 11 + [anch_spec] * 4,
        out_specs=[big, pl.BlockSpec((1, 4, AR, 128), lambda b: (b, 0, 0, 0)),
                   scal, scal, scal],
        out_shape=[jax.ShapeDtypeStruct((B, AR, 128), jnp.int32),
                   jax.ShapeDtypeStruct((B, 4, AR, 128), jnp.float32),
                   jax.ShapeDtypeStruct((1, 1), jnp.float32),
                   jax.ShapeDtypeStruct((1, 1), jnp.float32),
                   jax.ShapeDtypeStruct((1, 1), jnp.float32)],
        interpret=_INTERPRET,
    )(mi, *m, l0, l1, *p, *_AX)


def kernel(rpn_cls_logits, rpn_bbox_pred, gt_boxes, gt_labels, feat_map_shape):
    gtp = jnp.concatenate(
        [gt_boxes, jnp.zeros((B, GP - NG, 4), jnp.float32)], axis=1)
    g = [gtp[:, :, c].reshape(B, GP, 1, 1) for c in range(4)]
    vm = jnp.concatenate(
        [(gt_labels > 0).astype(jnp.float32),
         jnp.zeros((B, GP - NG), jnp.float32)], axis=1).reshape(B, GP, 1, 1)
    mi, m0, m1, m2, m3 = _stage1(g, vm)
    l0 = rpn_cls_logits[:, :, 0].reshape(B, AR, 128)
    l1 = rpn_cls_logits[:, :, 1].reshape(B, AR, 128)
    pred = rpn_bbox_pred.reshape(B, A, 4)
    p = [pred[:, :, c].reshape(B, AR, 128) for c in range(4)]
    lab3, tgt4, ce, slv, cnt = _stage2(mi, (m0, m1, m2, m3), l0, l1, p)
    cls_loss = ce[0, 0] / float(A * B)
    bbox_loss = slv[0, 0] / jnp.maximum(cnt[0, 0], 1.0)
    labels = lab3.reshape(B, A)
    targets = jnp.transpose(tgt4.reshape(B, 4, A), (0, 2, 1))
    return cls_loss, bbox_loss, labels, targets


# speedup vs baseline: 3.2926x; 1.2211x over previous
import functools

import jax
import jax.numpy as jnp
import numpy as np
from jax import lax
from jax.experimental import pallas as pl
from jax.experimental.pallas import tpu as pltpu
from jax.experimental.pallas import tpu_sc as plsc

NA = 9
FH = 64
FW = 64
B = 4
NG = 50
A = FH * FW * NA
AR = A // 128
GP = 64
R1 = 16
NJ = AR // R1

_INTERPRET = False


def _anchors_np():
    base_size = 16
    anchors = []
    cx = base_size / 2.0
    cy = base_size / 2.0
    for r in (0.5, 1.0, 2.0):
        for s in (8, 16, 32):
            area = float(base_size * s) ** 2
            w = np.sqrt(area / r)
            h = w * r
            anchors.append([cx - 0.5 * w, cy - 0.5 * h, cx + 0.5 * w, cy + 0.5 * h])
    base = np.array(anchors, dtype=np.float32)
    shift_x = np.arange(FW, dtype=np.float32) * 16.0
    shift_y = np.arange(FH, dtype=np.float32) * 16.0
    sx, sy = np.meshgrid(shift_x, shift_y, indexing="ij")
    shifts = np.stack([sx, sy, sx, sy], axis=-1).reshape(-1, 4).astype(np.float32)
    return (base[None, :, :] + shifts[:, None, :]).reshape(-1, 4).astype(np.float32)


_ANCH = _anchors_np()
_AX = [np.ascontiguousarray(_ANCH[:, c].reshape(AR, 128)) for c in range(4)]


def _s1_body(ax1r, ay1r, ax2r, ay2r, g0r, g1r, g2r, g3r, vmr,
             mir, m0r, m1r, m2r, m3r):
    ax1 = ax1r[...][None]
    ay1 = ay1r[...][None]
    ax2 = ax2r[...][None]
    ay2 = ay2r[...][None]
    gx1 = g0r[0]
    gy1 = g1r[0]
    gx2 = g2r[0]
    gy2 = g3r[0]
    vm = vmr[0]
    x1 = jnp.maximum(ax1, gx1)
    y1 = jnp.maximum(ay1, gy1)
    x2 = jnp.minimum(ax2, gx2)
    y2 = jnp.minimum(ay2, gy2)
    inter = jnp.maximum(0.0, x2 - x1) * jnp.maximum(0.0, y2 - y1)
    a1 = (ax2 - ax1) * (ay2 - ay1)
    a2 = (gx2 - gx1) * (gy2 - gy1)
    iou = inter / (a1 + a2 - inter + 1e-8)
    iou = iou * vm + (vm - 1.0)
    mx = jnp.max(iou, axis=0)
    it = jax.lax.broadcasted_iota(jnp.int32, (GP, R1, 128), 0)
    am = jnp.min(jnp.where(iou >= mx[None], it, GP), axis=0)
    oh = (it == am[None]).astype(jnp.float32)
    mir[0] = mx
    m0r[0] = jnp.sum(oh * gx1, axis=0)
    m1r[0] = jnp.sum(oh * gy1, axis=0)
    m2r[0] = jnp.sum(oh * gx2, axis=0)
    m3r[0] = jnp.sum(oh * gy2, axis=0)


def _stage1(g, vm):
    anch_spec = pl.BlockSpec((R1, 128), lambda b, j: (j, 0))
    gt_spec = pl.BlockSpec((1, GP, 1, 1), lambda b, j: (b, 0, 0, 0))
    out_spec = pl.BlockSpec((1, R1, 128), lambda b, j: (b, j, 0))
    shp = jax.ShapeDtypeStruct((B, AR, 128), jnp.float32)
    return pl.pallas_call(
        _s1_body,
        grid=(B, NJ),
        in_specs=[anch_spec] * 4 + [gt_spec] * 5,
        out_specs=[out_spec] * 5,
        out_shape=[shp] * 5,
        interpret=_INTERPRET,
    )(*_AX, *g, vm)


NW = 32
APW = A // NW
NCH = APW // 16
_AXF = [np.ascontiguousarray(_ANCH[:, c]) for c in range(4)]


def _sc_match(gt_pack):
    mesh = plsc.VectorSubcoreMesh(core_axis_name="c", subcore_axis_name="s")
    shp = jax.ShapeDtypeStruct((B * A,), jnp.float32)

    @functools.partial(
        pl.kernel,
        mesh=mesh,
        out_type=[shp] * 5,
        scratch_types=[pltpu.VMEM((APW,), jnp.float32)] * 4
        + [pltpu.VMEM((GP,), jnp.float32)] * 5
        + [pltpu.VMEM((APW,), jnp.float32)] * 5,
        compiler_params=pltpu.CompilerParams(needs_layout_passes=False),
    )
    def k(ax1h, ay1h, ax2h, ay2h, gth,
          mih, m0h, m1h, m2h, m3h,
          ax1v, ay1v, ax2v, ay2v, gx1v, gy1v, gx2v, gy2v, vmv,
          miv, m0v, m1v, m2v, m3v):
        wid = lax.axis_index("s") * 2 + lax.axis_index("c")
        base = wid * APW
        pltpu.sync_copy(ax1h.at[pl.ds(base, APW)], ax1v)
        pltpu.sync_copy(ay1h.at[pl.ds(base, APW)], ay1v)
        pltpu.sync_copy(ax2h.at[pl.ds(base, APW)], ax2v)
        pltpu.sync_copy(ay2h.at[pl.ds(base, APW)], ay2v)
        for b in range(B):
            pltpu.sync_copy(gth.at[pl.ds((b * 5 + 0) * GP, GP)], gx1v)
            pltpu.sync_copy(gth.at[pl.ds((b * 5 + 1) * GP, GP)], gy1v)
            pltpu.sync_copy(gth.at[pl.ds((b * 5 + 2) * GP, GP)], gx2v)
            pltpu.sync_copy(gth.at[pl.ds((b * 5 + 3) * GP, GP)], gy2v)
            pltpu.sync_copy(gth.at[pl.ds((b * 5 + 4) * GP, GP)], vmv)

            def chunk(ch):
                s = pl.ds(ch * 16, 16)
                ax1 = ax1v[s]
                ay1 = ay1v[s]
                ax2 = ax2v[s]
                ay2 = ay2v[s]
                area1 = (ax2 - ax1) * (ay2 - ay1)

                def gt_step(g, carry):
                    rmax, ridx = carry
                    gv = jnp.full((16,), g, jnp.int32)
                    gx1 = plsc.load_gather(gx1v, [gv])
                    gy1 = plsc.load_gather(gy1v, [gv])
                    gx2 = plsc.load_gather(gx2v, [gv])
                    gy2 = plsc.load_gather(gy2v, [gv])
                    vm = plsc.load_gather(vmv, [gv])
                    iw = jnp.maximum(0.0, jnp.minimum(ax2, gx2) - jnp.maximum(ax1, gx1))
                    ih = jnp.maximum(0.0, jnp.minimum(ay2, gy2) - jnp.maximum(ay1, gy1))
                    inter = iw * ih
                    a2 = (gx2 - gx1) * (gy2 - gy1)
                    iou = inter / (area1 + a2 - inter + 1e-8)
                    iou = iou * vm + (vm - 1.0)
                    upd = iou > rmax
                    rmax = jnp.where(upd, iou, rmax)
                    ridx = jnp.where(upd, gv, ridx)
                    return rmax, ridx

                rmax = jnp.full((16,), -2.0, jnp.float32)
                ridx = jnp.zeros((16,), jnp.int32)
                rmax, ridx = lax.fori_loop(0, NG, gt_step, (rmax, ridx))
                miv[s] = rmax
                m0v[s] = plsc.load_gather(gx1v, [ridx])
                m1v[s] = plsc.load_gather(gy1v, [ridx])
                m2v[s] = plsc.load_gather(gx2v, [ridx])
                m3v[s] = plsc.load_gather(gy2v, [ridx])

            lax.fori_loop(0, NCH, lambda ch, _: (chunk(ch), 0)[1], 0)
            pltpu.sync_copy(miv, mih.at[pl.ds(b * A + base, APW)])
            pltpu.sync_copy(m0v, m0h.at[pl.ds(b * A + base, APW)])
            pltpu.sync_copy(m1v, m1h.at[pl.ds(b * A + base, APW)])
            pltpu.sync_copy(m2v, m2h.at[pl.ds(b * A + base, APW)])
            pltpu.sync_copy(m3v, m3h.at[pl.ds(b * A + base, APW)])

    return k(*_AXF, gt_pack.reshape(-1))


def _smooth_l1(d):
    ad = jnp.abs(d)
    return jnp.where(ad < 1.0, 0.5 * d * d, ad - 0.5)


def _s2_body(mir, m0r, m1r, m2r, m3r, l0r, l1r, p0r, p1r, p2r, p3r,
             ax1r, ay1r, ax2r, ay2r, labr, tgtr, cer, slr, cntr):
    b = pl.program_id(0)
    mx = mir[0]
    lab = mx >= 0.7
    labf = lab.astype(jnp.float32)
    labr[0] = lab.astype(jnp.int32)
    ax1 = ax1r[...]
    ay1 = ay1r[...]
    ax2 = ax2r[...]
    ay2 = ay2r[...]
    bw = ax2 - ax1 + 1.0
    bh = ay2 - ay1 + 1.0
    bcx = ax1 + 0.5 * bw
    bcy = ay1 + 0.5 * bh
    m0 = m0r[0]
    m1 = m1r[0]
    m2 = m2r[0]
    m3 = m3r[0]
    gw = m2 - m0 + 1.0
    gh = m3 - m1 + 1.0
    gcx = m0 + 0.5 * gw
    gcy = m1 + 0.5 * gh
    t0 = (gcx - bcx) / bw
    t1 = (gcy - bcy) / bh
    t2 = jnp.log(gw / bw)
    t3 = jnp.log(gh / bh)
    tgtr[0, 0] = t0
    tgtr[0, 1] = t1
    tgtr[0, 2] = t2
    tgtr[0, 3] = t3
    l0 = l0r[0]
    l1 = l1r[0]
    mm = jnp.maximum(l0, l1)
    lse = mm + jnp.log(jnp.exp(l0 - mm) + jnp.exp(l1 - mm))
    ce_b = jnp.sum(lse - jnp.where(lab, l1, l0), keepdims=True)
    sl = (_smooth_l1(p0r[0] - t0) + _smooth_l1(p1r[0] - t1)
          + _smooth_l1(p2r[0] - t2) + _smooth_l1(p3r[0] - t3))
    sl_b = jnp.sum(sl * labf, keepdims=True)
    cnt_b = jnp.sum(labf, keepdims=True)

    @pl.when(b == 0)
    def _():
        cer[...] = jnp.zeros((1, 1), jnp.float32)
        slr[...] = jnp.zeros((1, 1), jnp.float32)
        cntr[...] = jnp.zeros((1, 1), jnp.float32)

    cer[...] += ce_b
    slr[...] += sl_b
    cntr[...] += cnt_b


def _stage2(mi, m, l0, l1, p):
    big = pl.BlockSpec((1, AR, 128), lambda b: (b, 0, 0))
    anch_spec = pl.BlockSpec((AR, 128), lambda b: (0, 0))
    scal = pl.BlockSpec((1, 1), lambda b: (0, 0))
    return pl.pallas_call(
        _s2_body,
        grid=(B,),
        in_specs=[big] * 11 + [anch_spec] * 4,
        out_specs=[big, pl.BlockSpec((1, 4, AR, 128), lambda b: (b, 0, 0, 0)),
                   scal, scal, scal],
        out_shape=[jax.ShapeDtypeStruct((B, AR, 128), jnp.int32),
                   jax.ShapeDtypeStruct((B, 4, AR, 128), jnp.float32),
                   jax.ShapeDtypeStruct((1, 1), jnp.float32),
                   jax.ShapeDtypeStruct((1, 1), jnp.float32),
                   jax.ShapeDtypeStruct((1, 1), jnp.float32)],
        interpret=_INTERPRET,
    )(mi, *m, l0, l1, *p, *_AX)


def kernel(rpn_cls_logits, rpn_bbox_pred, gt_boxes, gt_labels, feat_map_shape):
    gtp = jnp.concatenate(
        [gt_boxes, jnp.zeros((B, GP - NG, 4), jnp.float32)], axis=1)
    vmf = jnp.concatenate(
        [(gt_labels > 0).astype(jnp.float32),
         jnp.zeros((B, GP - NG), jnp.float32)], axis=1)
    gt_pack = jnp.concatenate(
        [jnp.transpose(gtp, (0, 2, 1)), vmf[:, None, :]], axis=1)
    mi, m0, m1, m2, m3 = _sc_match(gt_pack)
    mi = mi.reshape(B, AR, 128)
    m0 = m0.reshape(B, AR, 128)
    m1 = m1.reshape(B, AR, 128)
    m2 = m2.reshape(B, AR, 128)
    m3 = m3.reshape(B, AR, 128)
    l0 = rpn_cls_logits[:, :, 0].reshape(B, AR, 128)
    l1 = rpn_cls_logits[:, :, 1].reshape(B, AR, 128)
    pred = rpn_bbox_pred.reshape(B, A, 4)
    p = [pred[:, :, c].reshape(B, AR, 128) for c in range(4)]
    lab3, tgt4, ce, slv, cnt = _stage2(mi, (m0, m1, m2, m3), l0, l1, p)
    cls_loss = ce[0, 0] / float(A * B)
    bbox_loss = slv[0, 0] / jnp.maximum(cnt[0, 0], 1.0)
    labels = lab3.reshape(B, A)
    targets = jnp.transpose(tgt4.reshape(B, 4, A), (0, 2, 1))
    return cls_loss, bbox_loss, labels, targets
